# baseline (speedup 1.0000x reference)
import functools

import jax
import jax.numpy as jnp
from jax import lax
from jax.experimental import pallas as pl
from jax.experimental.pallas import tpu as pltpu
from jax.experimental.pallas import tpu_sc as plsc

B = 16384
F = 26
CARD = 100000
D = 32
FD = F * D
EPS = 1e-5

NC = 2
NS = 16
NW = NC * NS
BF = B * F
PER_W = BF // NW
CHUNK = 416
N_CHUNKS = PER_W // CHUNK
GROUPS = CHUNK // 16

R_ENT = 2600026
R_PAD = 650007


def _sc_gather(pt, idx4, sub):
    mesh = plsc.VectorSubcoreMesh(core_axis_name="c", subcore_axis_name="s")

    @functools.partial(
        pl.kernel,
        mesh=mesh,
        compiler_params=pltpu.CompilerParams(use_tc_tiling_on_sc=True,
                                             needs_layout_passes=False),
        out_type=jax.ShapeDtypeStruct((BF * D,), jnp.float32),
        scratch_types=[
            pltpu.VMEM((CHUNK,), jnp.int32),
            pltpu.VMEM((CHUNK,), jnp.int32),
            pltpu.VMEM((CHUNK, 128), jnp.float32),
            pltpu.VMEM((CHUNK * D,), jnp.float32),
            pltpu.SemaphoreType.DMA,
        ],
    )
    def k(pt_hbm, idx4_hbm, sub_hbm, out_hbm, idx4_v, sub_v, rows_v, out_v,
          sem):
        wid = lax.axis_index("s") * NC + lax.axis_index("c")
        base0 = wid * PER_W

        def chunk_body(c, _):
            base = base0 + c * CHUNK
            pltpu.sync_copy(idx4_hbm.at[pl.ds(base, CHUNK)], idx4_v)
            pltpu.sync_copy(sub_hbm.at[pl.ds(base, CHUNK)], sub_v)
            pltpu.async_copy(pt_hbm.at[idx4_v], rows_v, sem).wait()

            def group_body(g, _):
                k16 = g * 16 + lax.iota(jnp.int32, 16)
                sb = sub_v[pl.ds(g * 16, 16)]
                for j in range(D):
                    val = plsc.load_gather(rows_v, [k16, sb + j])
                    plsc.store_scatter(out_v, [k16 * D + j], val)
                return 0

            lax.fori_loop(0, GROUPS, group_body, 0)
            pltpu.sync_copy(out_v, out_hbm.at[pl.ds(base * D, CHUNK * D)])
            return 0

        lax.fori_loop(0, N_CHUNKS, chunk_body, 0)

    return k(pt, idx4, sub)


BT = 1024


def _tc_body(emb_ref, w0, w1, w2, w3, b_ref, g_ref, bt_ref, s_ref, e_ref,
             out_ref):
    hi = jax.lax.Precision.HIGHEST
    e = emb_ref[...]
    c0 = jnp.dot(e[:, 0:256], w0[...], precision=hi)
    c1 = jnp.dot(e[:, 256:512], w1[...], precision=hi)
    c2 = jnp.dot(e[:, 512:768], w2[...], precision=hi)
    c3 = jnp.dot(e[:, 768:832], w3[...], precision=hi)
    c = jnp.concatenate([c0, c1, c2, c3], axis=1) + b_ref[...]
    sq = c * c
    msq = jnp.dot(sq, s_ref[...], precision=hi)
    r = lax.rsqrt(msq + EPS)
    scale = jnp.dot(r, e_ref[...], precision=hi)
    out_ref[...] = c * scale * g_ref[...] + bt_ref[...]


def _tc_norm(emb2, w0, w1, w2, w3, b832, g832, bt832, S, E):
    grid = (B // BT,)
    full = lambda shape: pl.BlockSpec(shape, lambda i: (0, 0))
    return pl.pallas_call(
        _tc_body,
        grid=grid,
        in_specs=[
            pl.BlockSpec((BT, FD), lambda i: (i, 0)),
            full((256, 256)), full((256, 256)), full((256, 256)),
            full((64, 64)),
            full((1, FD)), full((1, FD)), full((1, FD)),
            full((FD, 128)), full((128, FD)),
        ],
        out_specs=pl.BlockSpec((BT, FD), lambda i: (i, 0)),
        out_shape=jax.ShapeDtypeStruct((B, FD), jnp.float32),
    )(emb2, w0, w1, w2, w3, b832, g832, bt832, S, E)


def kernel(x, tables, proj_W, proj_b, gamma, beta):
    offs = (jnp.arange(F, dtype=jnp.int32) * (CARD + 1))[None, :]
    ent = (jnp.clip(x, 0, CARD).astype(jnp.int32) + offs).reshape(-1)
    idx4 = ent >> 2
    sub = (ent & 3) << 5

    flat_tab = tables.reshape(F * (CARD + 1), D)
    pt = jnp.concatenate(
        [flat_tab, jnp.zeros((R_PAD * 4 - R_ENT, D), jnp.float32)]
    ).reshape(R_PAD, 4 * D)

    C = jnp.eye(D, dtype=jnp.float32) - jnp.full((D, D), 1.0 / D,
                                                 dtype=jnp.float32)
    Wc = jnp.matmul(proj_W, C)
    bc = jnp.matmul(proj_b, C)

    blkdiag = jax.scipy.linalg.block_diag
    w0 = blkdiag(*[Wc[f] for f in range(0, 8)])
    w1 = blkdiag(*[Wc[f] for f in range(8, 16)])
    w2 = blkdiag(*[Wc[f] for f in range(16, 24)])
    w3 = blkdiag(*[Wc[f] for f in range(24, 26)])
    b832 = bc.reshape(1, FD)
    g832 = jnp.tile(gamma, F)[None, :]
    bt832 = jnp.tile(beta, F)[None, :]

    d_ids = jnp.arange(FD, dtype=jnp.int32) // D
    S = (d_ids[:, None] == jnp.arange(128, dtype=jnp.int32)[None, :]
         ).astype(jnp.float32) / D
    E = (jnp.arange(128, dtype=jnp.int32)[:, None] == d_ids[None, :]
         ).astype(jnp.float32)

    emb = _sc_gather(pt, idx4, sub)
    emb2 = emb.reshape(B, FD)
    out2 = _tc_norm(emb2, w0, w1, w2, w3, b832, g832, bt832, S, E)
    return out2.reshape(B, F, D)


# speedup vs baseline: 22.0538x; 22.0538x over previous
import functools

import jax
import jax.numpy as jnp
from jax import lax
from jax.experimental import pallas as pl
from jax.experimental.pallas import tpu as pltpu
from jax.experimental.pallas import tpu_sc as plsc

B = 16384
F = 26
CARD = 100000
D = 32
FD = F * D
EPS = 1e-5
ROW = CARD + 1

NC = 2
NS = 16
NW = NC * NS
P_PER_W = FD // NW
HALF = B // 2


def _sc_gather(planes, idxT):
    mesh = plsc.VectorSubcoreMesh(core_axis_name="c", subcore_axis_name="s")

    @functools.partial(
        pl.kernel,
        mesh=mesh,
        compiler_params=pltpu.CompilerParams(use_tc_tiling_on_sc=True,
                                             needs_layout_passes=False),
        out_type=jax.ShapeDtypeStruct((FD, B), jnp.float32),
        scratch_types=[
            pltpu.VMEM((ROW,), jnp.float32),
            pltpu.VMEM((HALF,), jnp.int32),
            pltpu.VMEM((HALF,), jnp.float32),
        ],
    )
    def k(pl_hbm, idx_hbm, out_hbm, plane_v, idx_v, out_v):
        wid = lax.axis_index("s") * NC + lax.axis_index("c")
        p0 = wid * P_PER_W

        def plane_body(t, _):
            p = p0 + t
            f = p // D
            pltpu.sync_copy(pl_hbm.at[p], plane_v)

            def half_body(h, _):
                pltpu.sync_copy(idx_hbm.at[f, pl.ds(h * HALF, HALF)], idx_v)

                def group_body(g, _):
                    i16 = idx_v[pl.ds(g * 16, 16)]
                    out_v[pl.ds(g * 16, 16)] = plsc.load_gather(plane_v, [i16])
                    return 0

                lax.fori_loop(0, HALF // 16, group_body, 0)
                pltpu.sync_copy(out_v, out_hbm.at[p, pl.ds(h * HALF, HALF)])
                return 0

            lax.fori_loop(0, 2, half_body, 0)
            return 0

        lax.fori_loop(0, P_PER_W, plane_body, 0)

    return k(planes, idxT)


BT = 1024


def _tc_body(et_ref, w0, w1, w2, w3, b_ref, g_ref, bt_ref, s_ref, e_ref,
             out_ref):
    hi = jax.lax.Precision.HIGHEST
    dn = (((0,), (0,)), ((), ()))
    et = et_ref[...]
    c0 = lax.dot_general(et[0:256, :], w0[...], dn, precision=hi)
    c1 = lax.dot_general(et[256:512, :], w1[...], dn, precision=hi)
    c2 = lax.dot_general(et[512:768, :], w2[...], dn, precision=hi)
    c3 = lax.dot_general(et[768:832, :], w3[...], dn, precision=hi)
    c = jnp.concatenate([c0, c1, c2, c3], axis=1) + b_ref[...]
    sq = c * c
    msq = jnp.dot(sq, s_ref[...], precision=hi)
    r = lax.rsqrt(msq + EPS)
    scale = jnp.dot(r, e_ref[...], precision=hi)
    out_ref[...] = c * scale * g_ref[...] + bt_ref[...]


def _tc_norm(embT, w0, w1, w2, w3, b832, g832, bt832, S, E):
    grid = (B // BT,)
    full = lambda shape: pl.BlockSpec(shape, lambda i: (0, 0))
    return pl.pallas_call(
        _tc_body,
        grid=grid,
        in_specs=[
            pl.BlockSpec((FD, BT), lambda i: (0, i)),
            full((256, 256)), full((256, 256)), full((256, 256)),
            full((64, 64)),
            full((1, FD)), full((1, FD)), full((1, FD)),
            full((FD, 128)), full((128, FD)),
        ],
        out_specs=pl.BlockSpec((BT, FD), lambda i: (i, 0)),
        out_shape=jax.ShapeDtypeStruct((B, FD), jnp.float32),
    )(embT, w0, w1, w2, w3, b832, g832, bt832, S, E)


def kernel(x, tables, proj_W, proj_b, gamma, beta):
    idxT = jnp.clip(x, 0, CARD).astype(jnp.int32).T
    planes = tables.transpose(0, 2, 1).reshape(FD, ROW)

    C = jnp.eye(D, dtype=jnp.float32) - jnp.full((D, D), 1.0 / D,
                                                 dtype=jnp.float32)
    Wc = jnp.matmul(proj_W, C)
    bc = jnp.matmul(proj_b, C)

    blkdiag = jax.scipy.linalg.block_diag
    w0 = blkdiag(*[Wc[f] for f in range(0, 8)])
    w1 = blkdiag(*[Wc[f] for f in range(8, 16)])
    w2 = blkdiag(*[Wc[f] for f in range(16, 24)])
    w3 = blkdiag(*[Wc[f] for f in range(24, 26)])
    b832 = bc.reshape(1, FD)
    g832 = jnp.tile(gamma, F)[None, :]
    bt832 = jnp.tile(beta, F)[None, :]

    d_ids = jnp.arange(FD, dtype=jnp.int32) // D
    S = (d_ids[:, None] == jnp.arange(128, dtype=jnp.int32)[None, :]
         ).astype(jnp.float32) / D
    E = (jnp.arange(128, dtype=jnp.int32)[:, None] == d_ids[None, :]
         ).astype(jnp.float32)

    embT = _sc_gather(planes, idxT)
    out2 = _tc_norm(embT, w0, w1, w2, w3, b832, g832, bt832, S, E)
    return out2.reshape(B, F, D)


# speedup vs baseline: 29.3954x; 1.3329x over previous
import functools

import jax
import jax.numpy as jnp
from jax import lax
from jax.experimental import pallas as pl
from jax.experimental.pallas import tpu as pltpu
from jax.experimental.pallas import tpu_sc as plsc

B = 16384
F = 26
CARD = 100000
D = 32
FD = F * D
EPS = 1e-5
ROW = CARD + 1

NC = 2
NS = 16
NW = NC * NS
P_PER_W = FD // NW
HALF = B // 2


def _sc_gather(planes, idxT):
    mesh = plsc.VectorSubcoreMesh(core_axis_name="c", subcore_axis_name="s")

    @functools.partial(
        pl.kernel,
        mesh=mesh,
        compiler_params=pltpu.CompilerParams(use_tc_tiling_on_sc=True,
                                             needs_layout_passes=False),
        out_type=jax.ShapeDtypeStruct((FD, B), jnp.float32),
        scratch_types=[
            pltpu.VMEM((ROW,), jnp.float32),
            pltpu.VMEM((HALF,), jnp.int32),
            pltpu.VMEM((HALF,), jnp.float32),
        ],
    )
    def k(pl_hbm, idx_hbm, out_hbm, plane_v, idx_v, out_v):
        wid = lax.axis_index("s") * NC + lax.axis_index("c")
        p0 = wid * P_PER_W

        def plane_body(t, _):
            p = p0 + t
            f = p // D
            pltpu.sync_copy(pl_hbm.at[p], plane_v)

            def half_body(h, _):
                pltpu.sync_copy(idx_hbm.at[f, pl.ds(h * HALF, HALF)], idx_v)

                def group_body(g, _):
                    i16 = idx_v[pl.ds(g * 16, 16)]
                    out_v[pl.ds(g * 16, 16)] = plsc.load_gather(plane_v, [i16])
                    return 0

                lax.fori_loop(0, HALF // 16, group_body, 0)
                pltpu.sync_copy(out_v, out_hbm.at[p, pl.ds(h * HALF, HALF)])
                return 0

            lax.fori_loop(0, 2, half_body, 0)
            return 0

        lax.fori_loop(0, P_PER_W, plane_body, 0)

    return k(planes, idxT)


BT = 1024


def _tc_body(et_ref, w0, w1, w2, w3, b_ref, g_ref, bt_ref, s_ref, e_ref,
             out_ref):
    hi = jax.lax.Precision.DEFAULT
    dn = (((0,), (0,)), ((), ()))
    et = et_ref[...]
    c0 = lax.dot_general(et[0:256, :], w0[...], dn, precision=hi)
    c1 = lax.dot_general(et[256:512, :], w1[...], dn, precision=hi)
    c2 = lax.dot_general(et[512:768, :], w2[...], dn, precision=hi)
    c3 = lax.dot_general(et[768:832, :], w3[...], dn, precision=hi)
    c = jnp.concatenate([c0, c1, c2, c3], axis=1) + b_ref[...]
    sq = c * c
    msq = jnp.dot(sq, s_ref[...], precision=hi)
    r = lax.rsqrt(msq + EPS)
    scale = jnp.dot(r, e_ref[...], precision=hi)
    out_ref[...] = c * scale * g_ref[...] + bt_ref[...]


def _tc_norm(embT, w0, w1, w2, w3, b832, g832, bt832, S, E):
    grid = (B // BT,)
    full = lambda shape: pl.BlockSpec(shape, lambda i: (0, 0))
    return pl.pallas_call(
        _tc_body,
        grid=grid,
        in_specs=[
            pl.BlockSpec((FD, BT), lambda i: (0, i)),
            full((256, 256)), full((256, 256)), full((256, 256)),
            full((64, 64)),
            full((1, FD)), full((1, FD)), full((1, FD)),
            full((FD, 128)), full((128, FD)),
        ],
        out_specs=pl.BlockSpec((BT, FD), lambda i: (i, 0)),
        out_shape=jax.ShapeDtypeStruct((B, FD), jnp.float32),
    )(embT, w0, w1, w2, w3, b832, g832, bt832, S, E)


def kernel(x, tables, proj_W, proj_b, gamma, beta):
    idxT = jnp.clip(x, 0, CARD).astype(jnp.int32).T
    planes = tables.transpose(0, 2, 1).reshape(FD, ROW)

    C = jnp.eye(D, dtype=jnp.float32) - jnp.full((D, D), 1.0 / D,
                                                 dtype=jnp.float32)
    Wc = jnp.matmul(proj_W, C)
    bc = jnp.matmul(proj_b, C)

    blkdiag = jax.scipy.linalg.block_diag
    w0 = blkdiag(*[Wc[f] for f in range(0, 8)])
    w1 = blkdiag(*[Wc[f] for f in range(8, 16)])
    w2 = blkdiag(*[Wc[f] for f in range(16, 24)])
    w3 = blkdiag(*[Wc[f] for f in range(24, 26)])
    b832 = bc.reshape(1, FD)
    g832 = jnp.tile(gamma, F)[None, :]
    bt832 = jnp.tile(beta, F)[None, :]

    d_ids = jnp.arange(FD, dtype=jnp.int32) // D
    S = (d_ids[:, None] == jnp.arange(128, dtype=jnp.int32)[None, :]
         ).astype(jnp.float32) / D
    E = (jnp.arange(128, dtype=jnp.int32)[:, None] == d_ids[None, :]
         ).astype(jnp.float32)

    embT = _sc_gather(planes, idxT)
    out2 = _tc_norm(embT, w0, w1, w2, w3, b832, g832, bt832, S, E)
    return out2.reshape(B, F, D)
